# baseline (device time: 33843 ns/iter reference)
import jax
import jax.numpy as jnp
from jax import lax
from jax.experimental import pallas as pl
from jax.experimental.pallas import tpu as pltpu

N_DEV = 4
B = 2
SQ = 128
SKV = 128
HQ = 4
DH = 64
DM = 512
DQK = HQ * DH
BLK = 64
SKV_G = N_DEV * SKV


def kernel(x, Wq, K_ext, V_ext, Wo):
    k2 = K_ext.reshape(B, SKV, DQK)
    v2 = V_ext.reshape(B, SKV, DQK)

    def body(x_ref, wq_ref, k_ref, v_ref, wo_ref, out_ref,
             kg, vg, ksend, krecv, vsend, vrecv):
        my = lax.axis_index("i")
        left = lax.rem(my + (N_DEV - 1), N_DEV)
        right = lax.rem(my + 1, N_DEV)

        barrier = pltpu.get_barrier_semaphore()
        for nbr in (left, right):
            pl.semaphore_signal(
                barrier, inc=1,
                device_id=(nbr,), device_id_type=pl.DeviceIdType.MESH,
            )
        pl.semaphore_wait(barrier, 2)

        kg[pl.ds(my, 1)] = k_ref[...].reshape(1, B, SKV, DQK)
        vg[pl.ds(my, 1)] = v_ref[...].reshape(1, B, SKV, DQK)

        for h in range(N_DEV - 1):
            o = lax.rem(my + (N_DEV - h), N_DEV) if h else my
            rk = pltpu.make_async_remote_copy(
                src_ref=kg.at[o], dst_ref=kg.at[o],
                send_sem=ksend.at[h], recv_sem=krecv.at[h],
                device_id=(right,), device_id_type=pl.DeviceIdType.MESH,
            )
            rv = pltpu.make_async_remote_copy(
                src_ref=vg.at[o], dst_ref=vg.at[o],
                send_sem=vsend.at[h], recv_sem=vrecv.at[h],
                device_id=(right,), device_id_type=pl.DeviceIdType.MESH,
            )
            rk.start()
            rv.start()
            rk.wait()
            rv.wait()

        xl = x_ref[...]
        q = jnp.dot(
            xl.reshape(B * SQ, DM), wq_ref[...],
            preferred_element_type=jnp.float32,
        ).reshape(B, SQ, HQ, DH)

        ii = lax.broadcasted_iota(jnp.int32, (SQ, SKV_G), 0)
        jj = lax.broadcasted_iota(jnp.int32, (SQ, SKV_G), 1)
        qb = (my * SQ + ii) // BLK
        kb = jj // BLK
        mask = kb <= qb

        wo = wo_ref[...]
        for b in range(B):
            kslabs = [kg[s, b] for s in range(N_DEV)]
            vslabs = [vg[s, b] for s in range(N_DEV)]
            ctx_heads = []
            for hh in range(HQ):
                qbh = q[b, :, hh, :]
                s_parts = [
                    jnp.dot(
                        qbh, kslabs[s][:, hh * DH:(hh + 1) * DH].T,
                        preferred_element_type=jnp.float32,
                    )
                    for s in range(N_DEV)
                ]
                scores = jnp.concatenate(s_parts, axis=1) * 0.125
                scores = jnp.where(mask, scores, -1e9)
                m = jnp.max(scores, axis=1, keepdims=True)
                w = jnp.exp(scores - m)
                w = w / jnp.sum(w, axis=1, keepdims=True)
                ctx = sum(
                    jnp.dot(
                        w[:, s * SKV:(s + 1) * SKV],
                        vslabs[s][:, hh * DH:(hh + 1) * DH],
                        preferred_element_type=jnp.float32,
                    )
                    for s in range(N_DEV)
                )
                ctx_heads.append(ctx)
            ctx_b = jnp.concatenate(ctx_heads, axis=1)
            out_ref[b] = jnp.dot(
                ctx_b, wo, preferred_element_type=jnp.float32
            )

    return pl.pallas_call(
        body,
        out_shape=jax.ShapeDtypeStruct((B, SQ, DM), jnp.float32),
        in_specs=[pl.BlockSpec(memory_space=pltpu.VMEM)] * 5,
        out_specs=pl.BlockSpec(memory_space=pltpu.VMEM),
        scratch_shapes=[
            pltpu.VMEM((N_DEV, B, SKV, DQK), jnp.float32),
            pltpu.VMEM((N_DEV, B, SKV, DQK), jnp.float32),
            pltpu.SemaphoreType.DMA((N_DEV - 1,)),
            pltpu.SemaphoreType.DMA((N_DEV - 1,)),
            pltpu.SemaphoreType.DMA((N_DEV - 1,)),
            pltpu.SemaphoreType.DMA((N_DEV - 1,)),
        ],
        compiler_params=pltpu.CompilerParams(collective_id=0),
    )(x, Wq, k2, v2, Wo)


# device time: 24046 ns/iter; 1.4074x vs baseline; 1.4074x over previous
import jax
import jax.numpy as jnp
from jax import lax
from jax.experimental import pallas as pl
from jax.experimental.pallas import tpu as pltpu

N_DEV = 4
B = 2
SQ = 128
SKV = 128
HQ = 4
DH = 64
DM = 512
DQK = HQ * DH
BLK = 64
NEG = -1e9

A_K_R, A_V_R, A_K_L, A_V_L, B_K, B_V = range(6)


def kernel(x, Wq, K_ext, V_ext, Wo):
    k2 = K_ext.reshape(B, SKV, DQK)
    v2 = V_ext.reshape(B, SKV, DQK)

    def body(x_ref, wq_ref, k_ref, v_ref, wo_ref, out_ref,
             kg, vg, ss, rs):
        my = lax.axis_index("i")
        left = lax.rem(my + (N_DEV - 1), N_DEV)
        right = lax.rem(my + 1, N_DEV)

        def rc(src, dst, i, dev):
            return pltpu.make_async_remote_copy(
                src_ref=src, dst_ref=dst,
                send_sem=ss.at[i], recv_sem=rs.at[i],
                device_id=(dev,), device_id_type=pl.DeviceIdType.MESH,
            )

        barrier = pltpu.get_barrier_semaphore()
        for nbr in (left, right):
            pl.semaphore_signal(
                barrier, inc=1,
                device_id=(nbr,), device_id_type=pl.DeviceIdType.MESH,
            )
        pl.semaphore_wait(barrier, 2)

        a_kr = rc(k_ref, kg.at[my], A_K_R, right)
        a_vr = rc(v_ref, vg.at[my], A_V_R, right)
        a_kl = rc(k_ref, kg.at[my], A_K_L, left)
        a_vl = rc(v_ref, vg.at[my], A_V_L, left)
        a_kr.start()
        a_vr.start()
        a_kl.start()
        a_vl.start()

        q = jnp.dot(
            x_ref[...].reshape(B * SQ, DM), wq_ref[...],
            preferred_element_type=jnp.float32,
        ).reshape(B, SQ, HQ, DH)

        qb = (
            my * SQ + lax.broadcasted_iota(jnp.int32, (SQ, SKV), 0)
        ) // BLK
        jl = lax.broadcasted_iota(jnp.int32, (SQ, SKV), 1)

        state = [
            (jnp.full((SQ, 1), NEG, jnp.float32),
             jnp.zeros((SQ, 1), jnp.float32),
             jnp.zeros((SQ, DH), jnp.float32))
            for _ in range(B * HQ)
        ]

        def attend(state, s_idx, ka, va):
            kb = (s_idx * SKV + jl) // BLK
            msk = kb <= qb
            out = []
            for b in range(B):
                for hh in range(HQ):
                    m, l, acc = state[b * HQ + hh]
                    sc = jnp.dot(
                        q[b, :, hh, :],
                        ka[b][:, hh * DH:(hh + 1) * DH].T,
                        preferred_element_type=jnp.float32,
                    ) * 0.125
                    sc = jnp.where(msk, sc, NEG)
                    m2 = jnp.maximum(m, jnp.max(sc, axis=1, keepdims=True))
                    scale = jnp.exp(m - m2)
                    p = jnp.exp(sc - m2)
                    l2 = l * scale + jnp.sum(p, axis=1, keepdims=True)
                    acc2 = acc * scale + jnp.dot(
                        p, va[b][:, hh * DH:(hh + 1) * DH],
                        preferred_element_type=jnp.float32,
                    )
                    out.append((m2, l2, acc2))
            return out

        state = attend(state, my, k_ref[...], v_ref[...])

        s1 = lax.rem(my + (N_DEV - 1), N_DEV)
        rc(k_ref, kg.at[s1], A_K_R, left).wait_recv()
        rc(v_ref, vg.at[s1], A_V_R, left).wait_recv()

        b_k = rc(kg.at[s1], kg.at[s1], B_K, right)
        b_v = rc(vg.at[s1], vg.at[s1], B_V, right)
        b_k.start()
        b_v.start()

        state = attend(
            state, s1,
            kg[pl.ds(s1, 1)].reshape(B, SKV, DQK),
            vg[pl.ds(s1, 1)].reshape(B, SKV, DQK),
        )

        s2 = lax.rem(my + 1, N_DEV)
        rc(k_ref, kg.at[s2], A_K_L, right).wait_recv()
        rc(v_ref, vg.at[s2], A_V_L, right).wait_recv()
        state = attend(
            state, s2,
            kg[pl.ds(s2, 1)].reshape(B, SKV, DQK),
            vg[pl.ds(s2, 1)].reshape(B, SKV, DQK),
        )

        s3 = lax.rem(my + 2, N_DEV)
        rc(k_ref, kg.at[s3], B_K, left).wait_recv()
        rc(v_ref, vg.at[s3], B_V, left).wait_recv()
        state = attend(
            state, s3,
            kg[pl.ds(s3, 1)].reshape(B, SKV, DQK),
            vg[pl.ds(s3, 1)].reshape(B, SKV, DQK),
        )

        wo = wo_ref[...]
        for b in range(B):
            ctx_b = jnp.concatenate(
                [state[b * HQ + hh][2] / state[b * HQ + hh][1]
                 for hh in range(HQ)],
                axis=1,
            )
            out_ref[b] = jnp.dot(
                ctx_b, wo, preferred_element_type=jnp.float32
            )

        a_kr.wait_send()
        a_vr.wait_send()
        a_kl.wait_send()
        a_vl.wait_send()
        b_k.wait_send()
        b_v.wait_send()

    return pl.pallas_call(
        body,
        out_shape=jax.ShapeDtypeStruct((B, SQ, DM), jnp.float32),
        in_specs=[pl.BlockSpec(memory_space=pltpu.VMEM)] * 5,
        out_specs=pl.BlockSpec(memory_space=pltpu.VMEM),
        scratch_shapes=[
            pltpu.VMEM((N_DEV, B, SKV, DQK), jnp.float32),
            pltpu.VMEM((N_DEV, B, SKV, DQK), jnp.float32),
            pltpu.SemaphoreType.DMA((6,)),
            pltpu.SemaphoreType.DMA((6,)),
        ],
        compiler_params=pltpu.CompilerParams(collective_id=0),
    )(x, Wq, k2, v2, Wo)


# device time: 23360 ns/iter; 1.4488x vs baseline; 1.0294x over previous
import jax
import jax.numpy as jnp
from jax import lax
from jax.experimental import pallas as pl
from jax.experimental.pallas import tpu as pltpu

N_DEV = 4
B = 2
SQ = 128
SKV = 128
HQ = 4
DH = 64
DM = 512
DQK = HQ * DH
BLK = 64
NEG = -1e9

A_K_R, A_V_R, A_K_L, A_V_L, B_K, B_V = range(6)


def kernel(x, Wq, K_ext, V_ext, Wo):
    k2 = K_ext.reshape(B, SKV, DQK)
    v2 = V_ext.reshape(B, SKV, DQK)

    def body(x_ref, wq_ref, k_ref, v_ref, wo_ref, out_ref,
             kg, vg, ss, rs):
        my = lax.axis_index("i")
        left = lax.rem(my + (N_DEV - 1), N_DEV)
        right = lax.rem(my + 1, N_DEV)

        def rc(src, dst, i, dev):
            return pltpu.make_async_remote_copy(
                src_ref=src, dst_ref=dst,
                send_sem=ss.at[i], recv_sem=rs.at[i],
                device_id=(dev,), device_id_type=pl.DeviceIdType.MESH,
            )

        barrier = pltpu.get_barrier_semaphore()
        for nbr in (left, right):
            pl.semaphore_signal(
                barrier, inc=1,
                device_id=(nbr,), device_id_type=pl.DeviceIdType.MESH,
            )
        pl.semaphore_wait(barrier, 2)

        a_kr = rc(k_ref, kg.at[my], A_K_R, right)
        a_vr = rc(v_ref, vg.at[my], A_V_R, right)
        a_kl = rc(k_ref, kg.at[my], A_K_L, left)
        a_vl = rc(v_ref, vg.at[my], A_V_L, left)
        a_kr.start()
        a_vr.start()
        a_kl.start()
        a_vl.start()

        q = jnp.dot(
            x_ref[...].reshape(B * SQ, DM), wq_ref[...],
            preferred_element_type=jnp.float32,
        ).reshape(B, SQ, HQ, DH)

        qb = (
            my * SQ + lax.broadcasted_iota(jnp.int32, (SQ, SKV), 0)
        ) // BLK
        jl = lax.broadcasted_iota(jnp.int32, (SQ, SKV), 1)

        state = [
            (jnp.full((SQ, 1), NEG, jnp.float32),
             jnp.zeros((SQ, 1), jnp.float32),
             jnp.zeros((SQ, DH), jnp.float32))
            for _ in range(B * HQ)
        ]

        def attend(state, s_idx, ka, va):
            kb = (s_idx * SKV + jl) // BLK
            msk = kb <= qb
            out = []
            for b in range(B):
                for hh in range(HQ):
                    m, l, acc = state[b * HQ + hh]
                    sc = jnp.dot(
                        q[b, :, hh, :],
                        ka[b][:, hh * DH:(hh + 1) * DH].T,
                        preferred_element_type=jnp.float32,
                    ) * 0.125
                    sc = jnp.where(msk, sc, NEG)
                    m2 = jnp.maximum(m, jnp.max(sc, axis=1, keepdims=True))
                    scale = jnp.exp(m - m2)
                    p = jnp.exp(sc - m2)
                    l2 = l * scale + jnp.sum(p, axis=1, keepdims=True)
                    acc2 = acc * scale + jnp.dot(
                        p, va[b][:, hh * DH:(hh + 1) * DH],
                        preferred_element_type=jnp.float32,
                    )
                    out.append((m2, l2, acc2))
            return out

        state = attend(state, my, k_ref[...], v_ref[...])

        s1 = lax.rem(my + (N_DEV - 1), N_DEV)
        rc(k_ref, kg.at[s1], A_K_R, left).wait_recv()
        b_k = rc(kg.at[s1], kg.at[s1], B_K, right)
        b_k.start()
        rc(v_ref, vg.at[s1], A_V_R, left).wait_recv()

        state = attend(
            state, s1,
            kg[pl.ds(s1, 1)].reshape(B, SKV, DQK),
            vg[pl.ds(s1, 1)].reshape(B, SKV, DQK),
        )

        s2 = lax.rem(my + 1, N_DEV)
        rc(k_ref, kg.at[s2], A_K_L, right).wait_recv()
        rc(v_ref, vg.at[s2], A_V_L, right).wait_recv()
        b_v = rc(vg.at[s2], vg.at[s2], B_V, left)
        b_v.start()
        state = attend(
            state, s2,
            kg[pl.ds(s2, 1)].reshape(B, SKV, DQK),
            vg[pl.ds(s2, 1)].reshape(B, SKV, DQK),
        )

        s3 = lax.rem(my + 2, N_DEV)
        rc(k_ref, kg.at[s3], B_K, left).wait_recv()
        rc(v_ref, vg.at[s3], B_V, right).wait_recv()
        state = attend(
            state, s3,
            kg[pl.ds(s3, 1)].reshape(B, SKV, DQK),
            vg[pl.ds(s3, 1)].reshape(B, SKV, DQK),
        )

        wo = wo_ref[...]
        for b in range(B):
            ctx_b = jnp.concatenate(
                [state[b * HQ + hh][2] / state[b * HQ + hh][1]
                 for hh in range(HQ)],
                axis=1,
            )
            out_ref[b] = jnp.dot(
                ctx_b, wo, preferred_element_type=jnp.float32
            )

        a_kr.wait_send()
        a_vr.wait_send()
        a_kl.wait_send()
        a_vl.wait_send()
        b_k.wait_send()
        b_v.wait_send()

    return pl.pallas_call(
        body,
        out_shape=jax.ShapeDtypeStruct((B, SQ, DM), jnp.float32),
        in_specs=[pl.BlockSpec(memory_space=pltpu.VMEM)] * 5,
        out_specs=pl.BlockSpec(memory_space=pltpu.VMEM),
        scratch_shapes=[
            pltpu.VMEM((N_DEV, B, SKV, DQK), jnp.float32),
            pltpu.VMEM((N_DEV, B, SKV, DQK), jnp.float32),
            pltpu.SemaphoreType.DMA((6,)),
            pltpu.SemaphoreType.DMA((6,)),
        ],
        compiler_params=pltpu.CompilerParams(collective_id=0),
    )(x, Wq, k2, v2, Wo)
